# baseline (speedup 1.0000x reference)
import functools

import numpy as np
import jax
import jax.numpy as jnp
from jax import lax
from jax.experimental import pallas as pl
from jax.experimental.pallas import tpu as pltpu
from jax.experimental.pallas import tpu_sc as plsc

_B = 16384
_F = 26
_D = 32
_K = 5
_NUME = 13
_NC, _NS = 2, 16
_NW = _NC * _NS
_RPW = _B // _NW
_G = 4
_WIN = _G * _F
_NCHUNK = _RPW // _G
_NBUF = 2
_IPW = _RPW * _F
_P = 2
_RPP = _RPW // _P
_IPP = _RPP * _F
_NCHP = _RPP // _G

_OFFSETS = np.array((0, *np.cumsum([38462] * _F)[:-1]), dtype=np.int32)
_BLK = 2048


def _sc_body(emb_hbm, idx_hbm, s_hbm, q_hbm,
             idx_v, eb0, eb1, s_st, q_st, sem0, sem1):
    wid = lax.axis_index("c") * _NS + lax.axis_index("s")

    ebufs = (eb0, eb1)
    sems = (sem0, sem1)

    def issue(c, b):
        sl = idx_v.at[pl.ds(c * _WIN, _WIN)]
        pltpu.async_copy(emb_hbm.at[sl], ebufs[b], sems[b])

    def wait_chunk(c, b):
        sl = idx_v.at[pl.ds(c * _WIN, _WIN)]
        pltpu.make_async_copy(emb_hbm.at[sl], ebufs[b], sems[b]).wait()

    def process(c, b):
        eb = ebufs[b]
        for g in range(_G):
            row0 = g * _F
            acc0 = eb.at[row0, pl.ds(0, 16)][...]
            acc1 = eb.at[row0, pl.ds(16, 16)][...]
            q = acc0 * acc0 + acc1 * acc1
            for f in range(1, _F):
                e0 = eb.at[row0 + f, pl.ds(0, 16)][...]
                e1 = eb.at[row0 + f, pl.ds(16, 16)][...]
                acc0 = acc0 + e0
                acc1 = acc1 + e1
                q = q + e0 * e0 + e1 * e1
            r = c * _G + g
            s_st.at[r, pl.ds(0, 16)][...] = acc0
            s_st.at[r, pl.ds(16, 16)][...] = acc1
            q_st.at[r][...] = q

    for p in range(_P):
        pbase = wid * _RPW + p * _RPP
        pibase = wid * _IPW + p * _IPP
        pltpu.sync_copy(idx_hbm.at[pl.ds(pibase, _IPP)], idx_v)

        for b in range(_NBUF):
            issue(b, b)

        @pl.loop(0, _NCHP - _NBUF, step=_NBUF)
        def _(t):
            for b in range(_NBUF):
                c = t + b
                wait_chunk(c, b)
                process(c, b)
                issue(c + _NBUF, b)

        for b in range(_NBUF):
            c = _NCHP - _NBUF + b
            wait_chunk(c, b)
            process(c, b)

        pltpu.sync_copy(s_st, s_hbm.at[pl.ds(pbase, _RPP)])
        pltpu.sync_copy(q_st, q_hbm.at[pl.ds(pbase, _RPP)])


def _fc_body(fc_hbm, idx_hbm, lin_hbm, idx_v, fcs, lin_st, fsem):
    wid = lax.axis_index("c") * _NS + lax.axis_index("s")
    base = wid * _RPW
    ibase = wid * _IPW
    pltpu.sync_copy(idx_hbm.at[pl.ds(ibase, _IPW)], idx_v)

    @pl.loop(0, _NCHUNK)
    def _(c):
        sl = idx_v.at[pl.ds(c * _WIN, _WIN)]
        pltpu.async_copy(fc_hbm.at[sl], fcs.at[pl.ds(c * _WIN, _WIN)], fsem)

    @pl.loop(0, _NCHUNK)
    def _(c):
        sl = idx_v.at[pl.ds(c * _WIN, _WIN)]
        pltpu.make_async_copy(
            fc_hbm.at[sl], fcs.at[pl.ds(c * _WIN, _WIN)], fsem).wait()

    lane = jax.lax.iota(jnp.int32, 16) * _F

    @pl.loop(0, _RPW, step=16)
    def _(r0):
        fbase = r0 * _F
        lin = plsc.load_gather(fcs, [fbase + lane])
        for f in range(1, _F):
            lin = lin + plsc.load_gather(fcs, [fbase + f + lane])
        lin_st.at[pl.ds(r0, 16)][...] = lin

    pltpu.sync_copy(lin_st, lin_hbm.at[pl.ds(base, _RPW)])


def _sc_fc_lin(fc_flat, idx_flat):
    mesh = plsc.VectorSubcoreMesh(core_axis_name="c", subcore_axis_name="s")
    k = pl.kernel(
        _fc_body,
        compiler_params=pltpu.CompilerParams(
            use_tc_tiling_on_sc=False, needs_layout_passes=False),
        out_type=jax.ShapeDtypeStruct((_B,), jnp.float32),
        mesh=mesh,
        scratch_types=[
            pltpu.VMEM((_IPW,), jnp.int32),
            pltpu.VMEM((_IPW,), jnp.float32),
            pltpu.VMEM((_RPW,), jnp.float32),
            pltpu.SemaphoreType.DMA,
        ],
    )
    return k(fc_flat, idx_flat)


def _sc_gather_fm(emb_table, idx_flat):
    mesh = plsc.VectorSubcoreMesh(core_axis_name="c", subcore_axis_name="s")
    k = pl.kernel(
        _sc_body,
        compiler_params=pltpu.CompilerParams(
            use_tc_tiling_on_sc=False, needs_layout_passes=False),
        out_type=(
            jax.ShapeDtypeStruct((_B, _D), jnp.float32),
            jax.ShapeDtypeStruct((_B, 16), jnp.float32),
        ),
        mesh=mesh,
        scratch_types=[
            pltpu.VMEM((_IPP,), jnp.int32),
            pltpu.VMEM((_WIN, _D), jnp.float32),
            pltpu.VMEM((_WIN, _D), jnp.float32),
            pltpu.VMEM((_RPP, _D), jnp.float32),
            pltpu.VMEM((_RPP, 16), jnp.float32),
            pltpu.SemaphoreType.DMA,
            pltpu.SemaphoreType.DMA,
        ],
    )
    return k(emb_table, idx_flat)


def _combine_body(s_ref, q_ref, lin_ref, xd_ref, w_ref, v_ref, scal_ref, o_ref):
    s = s_ref[...]
    ix = 0.5 * (jnp.sum(s * s, axis=1, keepdims=True)
                - jnp.sum(q_ref[...], axis=1, keepdims=True))
    lin = lin_ref[...]
    xd = xd_ref[...]
    x1 = jnp.sum(xd * w_ref[...], axis=1, keepdims=True)
    xdsq = xd * xd
    vv = v_ref[...]
    acc = jnp.zeros_like(x1)
    for k in range(_K):
        vk = vv[k:k + 1, :]
        t = jnp.sum(xd * vk, axis=1, keepdims=True)
        t2 = jnp.sum(xdsq * (vk * vk), axis=1, keepdims=True)
        acc = acc + t * t - t2
    tot = ix + lin + x1 + 0.5 * acc + scal_ref[0]
    o_ref[...] = jax.nn.sigmoid(tot)


def _combine(s, q, lin, X_dense, num_W, v, scal):
    return pl.pallas_call(
        _combine_body,
        grid=(_B // _BLK,),
        out_shape=jax.ShapeDtypeStruct((_B, 1), jnp.float32),
        in_specs=[
            pl.BlockSpec((_BLK, _D), lambda i: (i, 0)),
            pl.BlockSpec((_BLK, 16), lambda i: (i, 0)),
            pl.BlockSpec((_BLK, 1), lambda i: (i, 0)),
            pl.BlockSpec((_BLK, _NUME), lambda i: (i, 0)),
            pl.BlockSpec((1, _NUME), lambda i: (0, 0)),
            pl.BlockSpec((_K, _NUME), lambda i: (0, 0)),
            pl.BlockSpec(memory_space=pltpu.SMEM),
        ],
        out_specs=pl.BlockSpec((_BLK, 1), lambda i: (i, 0)),
    )(s, q, lin, X_dense, num_W, v, scal)


def kernel(X_sparse, X_dense, emb_table, fc_table, bias, num_W, num_b, v):
    fc_flat = fc_table.reshape(fc_table.shape[0])
    idx = X_sparse + jnp.asarray(_OFFSETS)[None, :]
    idx_flat = idx.reshape(_B * _F)
    emb_lin = jax.lax.optimization_barrier(
        emb_table.reshape(-1)).reshape(emb_table.shape)
    s, q = _sc_gather_fm(emb_lin, idx_flat)
    lin = _sc_fc_lin(fc_flat, idx_flat)
    scal = (bias + num_b).astype(jnp.float32)
    out = _combine(s, q, lin.reshape(_B, 1), X_dense, num_W, v, scal)
    return jnp.squeeze(out, axis=1)


# speedup vs baseline: 1.0014x; 1.0014x over previous
import functools

import numpy as np
import jax
import jax.numpy as jnp
from jax import lax
from jax.experimental import pallas as pl
from jax.experimental.pallas import tpu as pltpu
from jax.experimental.pallas import tpu_sc as plsc

_B = 16384
_F = 26
_D = 32
_K = 5
_NUME = 13
_NC, _NS = 2, 16
_NW = _NC * _NS
_RPW = _B // _NW
_G = 4
_WIN = _G * _F
_NCHUNK = _RPW // _G
_NBUF = 2
_IPW = _RPW * _F
_P = 2
_RPP = _RPW // _P
_IPP = _RPP * _F
_NCHP = _RPP // _G

_OFFSETS = np.array((0, *np.cumsum([38462] * _F)[:-1]), dtype=np.int32)
_BLK = 2048


def _sc_body(emb_hbm, idx_hbm, s_hbm, q_hbm,
             idx_v, eb0, eb1, s_st, q_st, sem0, sem1):
    wid = lax.axis_index("c") * _NS + lax.axis_index("s")

    ebufs = (eb0, eb1)
    sems = (sem0, sem1)

    def issue(c, b):
        sl = idx_v.at[pl.ds(c * _WIN, _WIN)]
        pltpu.async_copy(emb_hbm.at[sl], ebufs[b], sems[b])

    def wait_chunk(c, b):
        sl = idx_v.at[pl.ds(c * _WIN, _WIN)]
        pltpu.make_async_copy(emb_hbm.at[sl], ebufs[b], sems[b]).wait()

    def process(c, b):
        eb = ebufs[b]
        for g in range(_G):
            row0 = g * _F
            acc0 = eb.at[row0, pl.ds(0, 16)][...]
            acc1 = eb.at[row0, pl.ds(16, 16)][...]
            q = acc0 * acc0 + acc1 * acc1
            for f in range(1, _F):
                e0 = eb.at[row0 + f, pl.ds(0, 16)][...]
                e1 = eb.at[row0 + f, pl.ds(16, 16)][...]
                acc0 = acc0 + e0
                acc1 = acc1 + e1
                q = q + e0 * e0 + e1 * e1
            r = c * _G + g
            s_st.at[r, pl.ds(0, 16)][...] = acc0
            s_st.at[r, pl.ds(16, 16)][...] = acc1
            q_st.at[r][...] = q

    for p in range(_P):
        pbase = wid * _RPW + p * _RPP
        pibase = wid * _IPW + p * _IPP
        pltpu.sync_copy(idx_hbm.at[pl.ds(pibase, _IPP)], idx_v)

        for b in range(_NBUF):
            issue(b, b)

        @pl.loop(0, _NCHP - _NBUF, step=_NBUF)
        def _(t):
            for b in range(_NBUF):
                c = t + b
                wait_chunk(c, b)
                process(c, b)
                issue(c + _NBUF, b)

        for b in range(_NBUF):
            c = _NCHP - _NBUF + b
            wait_chunk(c, b)
            process(c, b)

        pltpu.sync_copy(s_st, s_hbm.at[pl.ds(pbase, _RPP)])
        pltpu.sync_copy(q_st, q_hbm.at[pl.ds(pbase, _RPP)])


def _fc_body(fc_hbm, idx_hbm, lin_hbm, idx_v, fcs, lin_st, fsem):
    wid = lax.axis_index("c") * _NS + lax.axis_index("s")
    base = wid * _RPW
    ibase = wid * _IPW
    pltpu.sync_copy(idx_hbm.at[pl.ds(ibase, _IPW)], idx_v)

    @pl.loop(0, _NCHUNK)
    def _(c):
        sl = idx_v.at[pl.ds(c * _WIN, _WIN)]
        pltpu.async_copy(fc_hbm.at[sl], fcs.at[pl.ds(c * _WIN, _WIN)], fsem)

    @pl.loop(0, _NCHUNK)
    def _(c):
        sl = idx_v.at[pl.ds(c * _WIN, _WIN)]
        pltpu.make_async_copy(
            fc_hbm.at[sl], fcs.at[pl.ds(c * _WIN, _WIN)], fsem).wait()

    lane = jax.lax.iota(jnp.int32, 16) * _F

    @pl.loop(0, _RPW, step=16)
    def _(r0):
        fbase = r0 * _F
        lin = plsc.load_gather(fcs, [fbase + lane])
        for f in range(1, _F):
            lin = lin + plsc.load_gather(fcs, [fbase + f + lane])
        lin_st.at[pl.ds(r0, 16)][...] = lin

    pltpu.sync_copy(lin_st, lin_hbm.at[pl.ds(base, _RPW)])


def _sc_fc_lin(fc_flat, idx_flat):
    mesh = plsc.VectorSubcoreMesh(core_axis_name="c", subcore_axis_name="s")
    k = pl.kernel(
        _fc_body,
        compiler_params=pltpu.CompilerParams(
            use_tc_tiling_on_sc=False, needs_layout_passes=False),
        out_type=jax.ShapeDtypeStruct((_B,), jnp.float32),
        mesh=mesh,
        scratch_types=[
            pltpu.VMEM((_IPW,), jnp.int32),
            pltpu.VMEM((_IPW,), jnp.float32),
            pltpu.VMEM((_RPW,), jnp.float32),
            pltpu.SemaphoreType.DMA,
        ],
    )
    return k(fc_flat, idx_flat)


def _sc_gather_fm(emb_table, idx_flat):
    mesh = plsc.VectorSubcoreMesh(core_axis_name="c", subcore_axis_name="s")
    k = pl.kernel(
        _sc_body,
        compiler_params=pltpu.CompilerParams(
            use_tc_tiling_on_sc=False, needs_layout_passes=False),
        out_type=(
            jax.ShapeDtypeStruct((_B, _D), jnp.float32),
            jax.ShapeDtypeStruct((_B, 16), jnp.float32),
        ),
        mesh=mesh,
        scratch_types=[
            pltpu.VMEM((_IPP,), jnp.int32),
            pltpu.VMEM((_WIN, _D), jnp.float32),
            pltpu.VMEM((_WIN, _D), jnp.float32),
            pltpu.VMEM((_RPP, _D), jnp.float32),
            pltpu.VMEM((_RPP, 16), jnp.float32),
            pltpu.SemaphoreType.DMA,
            pltpu.SemaphoreType.DMA,
        ],
    )
    return k(emb_table, idx_flat)


def _combine_body(s_ref, q_ref, lin_ref, xd_ref, w_ref, v_ref, scal_ref, o_ref):
    s = s_ref[...]
    ix = 0.5 * (jnp.sum(s * s, axis=1, keepdims=True)
                - jnp.sum(q_ref[...], axis=1, keepdims=True))
    lin = lin_ref[...]
    xd = xd_ref[...]
    x1 = jnp.sum(xd * w_ref[...], axis=1, keepdims=True)
    xdsq = xd * xd
    vv = v_ref[...]
    acc = jnp.zeros_like(x1)
    for k in range(_K):
        vk = vv[k:k + 1, :]
        t = jnp.sum(xd * vk, axis=1, keepdims=True)
        t2 = jnp.sum(xdsq * (vk * vk), axis=1, keepdims=True)
        acc = acc + t * t - t2
    tot = ix + lin + x1 + 0.5 * acc + scal_ref[0]
    o_ref[...] = jax.nn.sigmoid(tot)


def _combine(s, q, lin, X_dense, num_W, v, scal):
    return pl.pallas_call(
        _combine_body,
        grid=(_B // _BLK,),
        out_shape=jax.ShapeDtypeStruct((_B, 1), jnp.float32),
        in_specs=[
            pl.BlockSpec((_BLK, _D), lambda i: (i, 0)),
            pl.BlockSpec((_BLK, 16), lambda i: (i, 0)),
            pl.BlockSpec((_BLK, 1), lambda i: (i, 0)),
            pl.BlockSpec((_BLK, _NUME), lambda i: (i, 0)),
            pl.BlockSpec((1, _NUME), lambda i: (0, 0)),
            pl.BlockSpec((_K, _NUME), lambda i: (0, 0)),
            pl.BlockSpec(memory_space=pltpu.SMEM),
        ],
        out_specs=pl.BlockSpec((_BLK, 1), lambda i: (i, 0)),
    )(s, q, lin, X_dense, num_W, v, scal)


def kernel(X_sparse, X_dense, emb_table, fc_table, bias, num_W, num_b, v):
    fc_flat = fc_table.reshape(fc_table.shape[0])
    idx = X_sparse + jnp.asarray(_OFFSETS)[None, :]
    idx_flat = idx.reshape(_B * _F)
    s, q = _sc_gather_fm(emb_table, idx_flat)
    lin = _sc_fc_lin(fc_flat, idx_flat)
    scal = (bias + num_b).astype(jnp.float32)
    out = _combine(s, q, lin.reshape(_B, 1), X_dense, num_W, v, scal)
    return jnp.squeeze(out, axis=1)
